# baseline (speedup 1.0000x reference)
import functools

import jax
import jax.numpy as jnp
from jax import lax
from jax.experimental import pallas as pl
from jax.experimental.pallas import tpu as pltpu
from jax.experimental.pallas import tpu_sc as plsc

B = 16384
D = 128
NC = 2
NS = 16
NW = NC * NS
BPW = B // NW
CHUNKS = (256, 256)
BLK = 16
CG = D // 16


def _bcast_lane(v, l):
    return lax.gather(
        v,
        jnp.full((16, 1), l, jnp.int32),
        lax.GatherDimensionNumbers(
            offset_dims=(), collapsed_slice_dims=(0,), start_index_map=(0,)
        ),
        slice_sizes=(1,),
        mode=lax.GatherScatterMode.PROMISE_IN_BOUNDS,
    )


def _make_sc_kernel():
    mesh = plsc.VectorSubcoreMesh(core_axis_name="c", subcore_axis_name="s")

    @functools.partial(
        pl.kernel,
        mesh=mesh,
        compiler_params=pltpu.CompilerParams(needs_layout_passes=False),
        out_type=jax.ShapeDtypeStruct((NW, BPW, D), jnp.float32),
        scratch_types=[
            pltpu.VMEM((BPW,), jnp.int32),
            pltpu.VMEM((3, D), jnp.float32),
            pltpu.VMEM((BPW, D), jnp.float32),
            pltpu.SemaphoreType.DMA,
            pltpu.SemaphoreType.DMA,
        ],
    )
    def k(idx_hbm, table_hbm, out_hbm, idx_v, table_v, rows_v, sem, sem_in):
        wid = lax.axis_index("s") * NC + lax.axis_index("c")
        c_tab = pltpu.async_copy(table_hbm, table_v, sem_in)
        c_idx = pltpu.async_copy(idx_hbm.at[wid], idx_v, sem_in)
        c_tab.wait()
        c_idx.wait()

        tv = [[table_v[r, pl.ds(c * 16, 16)] for c in range(CG)]
              for r in range(3)]

        copies = []
        off = 0
        for chunk in CHUNKS:
            start = off

            def blk_body(ib, start=start):
                base = start + ib * BLK
                idxv = idx_v[pl.ds(base, BLK)]
                for l in range(BLK):
                    b = _bcast_lane(idxv, l)
                    m0 = b == 0
                    m1 = b == 1
                    for c in range(CG):
                        val = jnp.where(
                            m0, tv[0][c], jnp.where(m1, tv[1][c], tv[2][c])
                        )
                        rows_v[base + l, pl.ds(c * 16, 16)] = val

            pl.loop(0, chunk // BLK, unroll=True)(blk_body)
            copies.append(
                pltpu.async_copy(
                    rows_v.at[pl.ds(start, chunk)],
                    out_hbm.at[wid, pl.ds(start, chunk)],
                    sem,
                )
            )
            off += chunk
        for c in copies:
            c.wait()

    return k


_sc_kernel = _make_sc_kernel()


def kernel(state, state_embed):
    idx = state.reshape(NW, BPW)
    out = _sc_kernel(idx, state_embed)
    return out.reshape(B, D)


# speedup vs baseline: 1.4175x; 1.4175x over previous
import functools

import jax
import jax.numpy as jnp
from jax import lax
from jax.experimental import pallas as pl
from jax.experimental.pallas import tpu as pltpu
from jax.experimental.pallas import tpu_sc as plsc

B = 16384
D = 128
NC = 2
NS = 16
NW = NC * NS
BPW = B // NW
CHUNKS = (256, 256)
BLK = 16
CG = D // 16


def _bcast_lane(v, l):
    return lax.gather(
        v,
        jnp.full((16, 1), l, jnp.int32)
        if isinstance(l, int)
        else jnp.broadcast_to(l, (16,)).reshape(16, 1).astype(jnp.int32),
        lax.GatherDimensionNumbers(
            offset_dims=(), collapsed_slice_dims=(0,), start_index_map=(0,)
        ),
        slice_sizes=(1,),
        mode=lax.GatherScatterMode.PROMISE_IN_BOUNDS,
    )


def _make_sc_kernel():
    mesh = plsc.VectorSubcoreMesh(core_axis_name="c", subcore_axis_name="s")

    @functools.partial(
        pl.kernel,
        mesh=mesh,
        compiler_params=pltpu.CompilerParams(needs_layout_passes=False),
        out_type=jax.ShapeDtypeStruct((NW, BPW, D), jnp.float32),
        scratch_types=[
            pltpu.VMEM((BPW,), jnp.int32),
            pltpu.VMEM((3, D), jnp.float32),
            pltpu.VMEM((BPW, D), jnp.float32),
            pltpu.SemaphoreType.DMA,
            pltpu.SemaphoreType.DMA,
        ],
    )
    def k(idx_hbm, table_hbm, out_hbm, idx_v, table_v, rows_v, sem, sem_in):
        wid = lax.axis_index("s") * NC + lax.axis_index("c")
        c_tab = pltpu.async_copy(table_hbm, table_v, sem_in)
        c_idx = pltpu.async_copy(idx_hbm.at[wid], idx_v, sem_in)
        c_tab.wait()
        c_idx.wait()

        tv = [[table_v[r, pl.ds(c * 16, 16)] for c in range(CG)]
              for r in range(3)]

        copies = []
        off = 0
        for chunk in CHUNKS:
            start = off

            def blk_body(ib, start=start):
                base = start + ib * BLK
                idxv = idx_v[pl.ds(base, BLK)]

                def row_body(l):
                    b = _bcast_lane(idxv, l)
                    m0 = b == 0
                    m1 = b == 1
                    for c in range(CG):
                        val = jnp.where(
                            m0, tv[0][c], jnp.where(m1, tv[1][c], tv[2][c])
                        )
                        rows_v[base + l, pl.ds(c * 16, 16)] = val

                pl.loop(0, BLK, unroll=2)(row_body)

            pl.loop(0, chunk // BLK)(blk_body)
            copies.append(
                pltpu.async_copy(
                    rows_v.at[pl.ds(start, chunk)],
                    out_hbm.at[wid, pl.ds(start, chunk)],
                    sem,
                )
            )
            off += chunk
        for c in copies:
            c.wait()

    return k


_sc_kernel = _make_sc_kernel()


def kernel(state, state_embed):
    idx = state.reshape(NW, BPW)
    out = _sc_kernel(idx, state_embed)
    return out.reshape(B, D)
